# baseline (speedup 1.0000x reference)
import functools

import jax
import jax.numpy as jnp
from jax import lax
from jax.experimental import pallas as pl
from jax.experimental.pallas import tpu as pltpu
from jax.experimental.pallas import tpu_sc as plsc

_VOCAB = 1_000_000
_CAT_VOCAB = 1000
_DIM = 64
_BATCH = 16384
_IN_DIM = _DIM + _CAT_VOCAB + 1

_NC = 2
_NS = 16
_NW = _NC * _NS
_BPW = _BATCH // _NW
_CHUNK = 128
_NCHUNK = _BPW // _CHUNK


def _sc_gather_body(pk_idx_hbm, cat_idx_hbm, pk_table_hbm, w1_hbm,
                    pk_out, cat_out,
                    pk_idx_v, cat_idx_v, pk_rows, cat_rows, sem_pk, sem_cat):
    wid = lax.axis_index("s") * _NC + lax.axis_index("c")
    base = wid * _BPW
    pltpu.sync_copy(pk_idx_hbm.at[pl.ds(base, _BPW)], pk_idx_v)
    pltpu.sync_copy(cat_idx_hbm.at[pl.ds(base, _BPW)], cat_idx_v)
    for j in range(_BPW // 16):
        sl = pl.ds(j * 16, 16)
        cat_idx_v[sl] = cat_idx_v[sl] + _DIM
    copies = []
    for j in range(_NCHUNK):
        sl = pl.ds(j * _CHUNK, _CHUNK)
        copies.append(pltpu.async_copy(
            pk_table_hbm.at[pk_idx_v.at[sl]], pk_rows.at[sl], sem_pk))
        copies.append(pltpu.async_copy(
            w1_hbm.at[cat_idx_v.at[sl]], cat_rows.at[sl], sem_cat))
    for c in copies:
        c.wait()
    pltpu.sync_copy(pk_rows, pk_out.at[pl.ds(base, _BPW)])
    pltpu.sync_copy(cat_rows, cat_out.at[pl.ds(base, _BPW)])


@jax.jit
def _sc_gather(pk_idx, cat_idx, pk_table, w1):
    mesh = plsc.VectorSubcoreMesh(core_axis_name="c", subcore_axis_name="s")
    return pl.kernel(
        _sc_gather_body,
        out_type=[
            jax.ShapeDtypeStruct((_BATCH, _DIM), jnp.float32),
            jax.ShapeDtypeStruct((_BATCH, _DIM), jnp.float32),
        ],
        mesh=mesh,
        scratch_types=[
            pltpu.VMEM((_BPW,), jnp.int32),
            pltpu.VMEM((_BPW,), jnp.int32),
            pltpu.VMEM((_BPW, _DIM), jnp.float32),
            pltpu.VMEM((_BPW, _DIM), jnp.float32),
            pltpu.SemaphoreType.DMA,
            pltpu.SemaphoreType.DMA,
        ],
        compiler_params=pltpu.CompilerParams(use_tc_tiling_on_sc=False),
    )(pk_idx, cat_idx, pk_table, w1)


_BLK = 2048


def _fnn_body(pk_ref, cat_ref, num_ref, w1a_ref, w1n_ref, b1_ref, w2_ref,
              b2_ref, out_ref):
    h = lax.dot_general(pk_ref[...], w1a_ref[...], (((1,), (0,)), ((), ())),
                        precision=lax.Precision.HIGHEST,
                        preferred_element_type=jnp.float32)
    h = h + cat_ref[...] + num_ref[...] * w1n_ref[...] + b1_ref[...]
    h = jnp.maximum(h, 0.0)
    out_ref[...] = lax.dot_general(h, w2_ref[...], (((1,), (0,)), ((), ())),
                                   precision=lax.Precision.HIGHEST,
                                   preferred_element_type=jnp.float32) \
        + b2_ref[...]


@jax.jit
def _tc_fnn(pk_emb, cat_rows, num2, w1a, w1n, b1r, w2, b2r):
    grid = (_BATCH // _BLK,)
    return pl.pallas_call(
        _fnn_body,
        grid=grid,
        in_specs=[
            pl.BlockSpec((_BLK, _DIM), lambda i: (i, 0)),
            pl.BlockSpec((_BLK, _DIM), lambda i: (i, 0)),
            pl.BlockSpec((_BLK, 1), lambda i: (i, 0)),
            pl.BlockSpec((_DIM, _DIM), lambda i: (0, 0)),
            pl.BlockSpec((1, _DIM), lambda i: (0, 0)),
            pl.BlockSpec((1, _DIM), lambda i: (0, 0)),
            pl.BlockSpec((_DIM, _DIM), lambda i: (0, 0)),
            pl.BlockSpec((1, _DIM), lambda i: (0, 0)),
        ],
        out_specs=pl.BlockSpec((_BLK, _DIM), lambda i: (i, 0)),
        out_shape=jax.ShapeDtypeStruct((_BATCH, _DIM), jnp.float32),
        compiler_params=pltpu.CompilerParams(
            dimension_semantics=("arbitrary",)),
    )(pk_emb, cat_rows, num2, w1a, w1n, b1r, w2, b2r)


def kernel(pk_idx, cat_idx, num_feat, pk_table, W1, b1, W2, b2):
    pk_emb, cat_rows = _sc_gather(pk_idx, cat_idx, pk_table, W1)
    out = _tc_fnn(pk_emb, cat_rows,
                  num_feat.reshape(_BATCH, 1),
                  W1[:_DIM],
                  W1[_IN_DIM - 1:].reshape(1, _DIM),
                  b1.reshape(1, _DIM),
                  W2,
                  b2.reshape(1, _DIM))
    return out
